# Initial kernel scaffold; baseline (speedup 1.0000x reference)
import jax
import jax.numpy as jnp
from jax.experimental import pallas as pl


def kernel(mapping, lm, landmarks):
    raise NotImplementedError("write your pallas kernel here")


# speedup vs baseline: 6.2339x; 6.2339x over previous
import functools

import jax
import jax.numpy as jnp
from jax import lax
from jax.experimental import pallas as pl
from jax.experimental.pallas import tpu as pltpu
from jax.experimental.pallas import tpu_sc as plsc

_NC = 2
_NS = 16
_NW = _NC * _NS

_LANES = 16
_NUM_LM = 68
_LM_PAD = 80
_N_CHUNKS = _LM_PAD // _LANES
_VALID_TAIL = _NUM_LM - (_N_CHUNKS - 1) * _LANES


def _loss_kernel(n_rows, v, flat_hbm, lm_hbm, l0_hbm, l1_hbm,
                 out_hbm, lm_v, l0_v, l1_v, idx_v, vals_v, acc_v, sem):
    rows_per_w = n_rows // _NW
    wid = lax.axis_index("s") * _NC + lax.axis_index("c")
    base = wid * rows_per_w

    pltpu.sync_copy(lm_hbm, lm_v)
    pltpu.sync_copy(l0_hbm, l0_v)
    pltpu.sync_copy(l1_hbm, l1_v)

    def build_row(r, carry):
        row_off = (base + r) * v
        for k in range(_N_CHUNKS):
            chunk = lm_v[pl.ds(k * _LANES, _LANES)]
            idx_v[r, pl.ds(k * _LANES, _LANES)] = chunk + row_off
        return carry

    lax.fori_loop(0, rows_per_w, build_row, 0, unroll=False)

    def gather_row(r, carry):
        pltpu.async_copy(flat_hbm.at[idx_v.at[r]], vals_v.at[r], sem).wait()
        return carry

    lax.fori_loop(0, rows_per_w, gather_row, 0, unroll=False)

    lane = lax.iota(jnp.int32, _LANES)

    def reduce_row(r, acc):
        is_c1 = (r % 2) == 1
        for k in range(_N_CHUNKS):
            vals = vals_v[r, pl.ds(k * _LANES, _LANES)]
            tgt0 = l0_v[pl.ds(k * _LANES, _LANES)]
            tgt1 = l1_v[pl.ds(k * _LANES, _LANES)]
            tgt = jnp.where(is_c1, tgt1, tgt0)
            d = vals - tgt
            sq = d * d
            if k == _N_CHUNKS - 1:
                sq = jnp.where(lane < _VALID_TAIL, sq, 0.0)
            acc = acc + sq
        return acc

    acc = lax.fori_loop(0, rows_per_w, reduce_row,
                        jnp.zeros((_LANES,), jnp.float32), unroll=False)
    acc_v[...] = acc
    pltpu.sync_copy(acc_v, out_hbm.at[wid])


def kernel(mapping, lm, landmarks):
    n, two, v = mapping.shape
    num_lm = lm.shape[0]
    n_rows = n * two

    lm32 = lm.astype(jnp.int32)
    lm_pad = jnp.full((_LM_PAD,), lm32[0], jnp.int32).at[:num_lm].set(lm32)
    l0 = jnp.zeros((_LM_PAD,), jnp.float32).at[:num_lm].set(landmarks[:, 0])
    l1 = jnp.zeros((_LM_PAD,), jnp.float32).at[:num_lm].set(landmarks[:, 1])
    flat = mapping.reshape(-1)
    rows_per_w = n_rows // _NW

    mesh = plsc.VectorSubcoreMesh(core_axis_name="c", subcore_axis_name="s",
                                  num_cores=_NC, num_subcores=_NS)
    partials = pl.kernel(
        functools.partial(_loss_kernel, n_rows, v),
        out_type=jax.ShapeDtypeStruct((_NW, _LANES), jnp.float32),
        mesh=mesh,
        scratch_types=[
            pltpu.VMEM((_LM_PAD,), jnp.int32),
            pltpu.VMEM((_LM_PAD,), jnp.float32),
            pltpu.VMEM((_LM_PAD,), jnp.float32),
            pltpu.VMEM((rows_per_w, _LM_PAD), jnp.int32),
            pltpu.VMEM((rows_per_w, _LM_PAD), jnp.float32),
            pltpu.VMEM((_LANES,), jnp.float32),
            pltpu.SemaphoreType.DMA,
        ],
    )(flat, lm_pad, l0, l1)
    return jnp.sum(partials) / n


# speedup vs baseline: 7.0390x; 1.1291x over previous
import functools

import jax
import jax.numpy as jnp
from jax import lax
from jax.experimental import pallas as pl
from jax.experimental.pallas import tpu as pltpu
from jax.experimental.pallas import tpu_sc as plsc

_NC = 2
_NS = 16
_NW = _NC * _NS

_LANES = 16
_NUM_LM = 68
_LM_PAD = 80
_N_CHUNKS = _LM_PAD // _LANES
_VALID_TAIL = _NUM_LM - (_N_CHUNKS - 1) * _LANES


def _loss_kernel(n_rows, v, flat_hbm, lm_hbm, l0_hbm, l1_hbm,
                 out_hbm, lm_v, l0_v, l1_v, idx_v, vals_v, acc_v, sem):
    rows_per_w = n_rows // _NW
    wid = lax.axis_index("s") * _NC + lax.axis_index("c")
    base = wid * rows_per_w

    pltpu.sync_copy(lm_hbm, lm_v)
    pltpu.sync_copy(l0_hbm, l0_v)
    pltpu.sync_copy(l1_hbm, l1_v)

    def build_row(r, carry):
        row_off = (base + r) * v
        for k in range(_N_CHUNKS):
            chunk = lm_v[pl.ds(k * _LANES, _LANES)]
            idx_v[r, pl.ds(k * _LANES, _LANES)] = chunk + row_off
        return carry

    lax.fori_loop(0, rows_per_w, build_row, 0, unroll=False)

    def fire_row(r, carry):
        pltpu.async_copy(flat_hbm.at[idx_v.at[r]], vals_v.at[r], sem)
        return carry

    lax.fori_loop(0, rows_per_w, fire_row, 0, unroll=False)

    def drain_row(r, carry):
        pltpu.make_async_copy(flat_hbm.at[idx_v.at[r]], vals_v.at[r],
                              sem).wait()
        return carry

    lax.fori_loop(0, rows_per_w, drain_row, 0, unroll=False)

    lane = lax.iota(jnp.int32, _LANES)

    def reduce_row(r, acc):
        is_c1 = (r % 2) == 1
        for k in range(_N_CHUNKS):
            vals = vals_v[r, pl.ds(k * _LANES, _LANES)]
            tgt0 = l0_v[pl.ds(k * _LANES, _LANES)]
            tgt1 = l1_v[pl.ds(k * _LANES, _LANES)]
            tgt = jnp.where(is_c1, tgt1, tgt0)
            d = vals - tgt
            sq = d * d
            if k == _N_CHUNKS - 1:
                sq = jnp.where(lane < _VALID_TAIL, sq, 0.0)
            acc = acc + sq
        return acc

    acc = lax.fori_loop(0, rows_per_w, reduce_row,
                        jnp.zeros((_LANES,), jnp.float32), unroll=False)
    acc_v[...] = acc
    pltpu.sync_copy(acc_v, out_hbm.at[wid])


def kernel(mapping, lm, landmarks):
    n, two, v = mapping.shape
    num_lm = lm.shape[0]
    n_rows = n * two

    lm32 = lm.astype(jnp.int32)
    lm_pad = jnp.full((_LM_PAD,), lm32[0], jnp.int32).at[:num_lm].set(lm32)
    l0 = jnp.zeros((_LM_PAD,), jnp.float32).at[:num_lm].set(landmarks[:, 0])
    l1 = jnp.zeros((_LM_PAD,), jnp.float32).at[:num_lm].set(landmarks[:, 1])
    flat = mapping.reshape(-1)
    rows_per_w = n_rows // _NW

    mesh = plsc.VectorSubcoreMesh(core_axis_name="c", subcore_axis_name="s",
                                  num_cores=_NC, num_subcores=_NS)
    partials = pl.kernel(
        functools.partial(_loss_kernel, n_rows, v),
        out_type=jax.ShapeDtypeStruct((_NW, _LANES), jnp.float32),
        mesh=mesh,
        scratch_types=[
            pltpu.VMEM((_LM_PAD,), jnp.int32),
            pltpu.VMEM((_LM_PAD,), jnp.float32),
            pltpu.VMEM((_LM_PAD,), jnp.float32),
            pltpu.VMEM((rows_per_w, _LM_PAD), jnp.int32),
            pltpu.VMEM((rows_per_w, _LM_PAD), jnp.float32),
            pltpu.VMEM((_LANES,), jnp.float32),
            pltpu.SemaphoreType.DMA,
        ],
    )(flat, lm_pad, l0, l1)
    return jnp.sum(partials) / n


# speedup vs baseline: 72.3417x; 10.2773x over previous
import functools

import jax
import jax.numpy as jnp
from jax import lax
from jax.experimental import pallas as pl
from jax.experimental.pallas import tpu as pltpu
from jax.experimental.pallas import tpu_sc as plsc

_NC = 2
_NS = 16
_NW = _NC * _NS

_LANES = 16
_NUM_LM = 68
_LM_PAD = 80
_N_CHUNKS = _LM_PAD // _LANES
_NB = 128
_JH_CHUNKS = 3
_ROWS = _JH_CHUNKS * _LANES

_GATHER_DNUMS = lax.GatherDimensionNumbers(
    offset_dims=(), collapsed_slice_dims=(0,), start_index_map=(0,))


def _splat(vec, idx_scalar):
    rr = jnp.full((_LANES, 1), idx_scalar, jnp.int32)
    return lax.gather(vec, rr, _GATHER_DNUMS, slice_sizes=(1,),
                      mode=lax.GatherScatterMode.PROMISE_IN_BOUNDS)


def _loss_kernel(n, v, m2_hbm, lm_hbm, l0_hbm, l1_hbm,
                 out_hbm, lm_v, l0_v, l1_v, idx_v, vals_v, acc_v, sem):
    wid = lax.axis_index("s") * _NC + lax.axis_index("c")
    nt = wid // 4
    ch = (wid // 2) % 2
    jh = wid % 2
    is_c1 = ch == 1
    is_h1 = jh == 1

    pltpu.sync_copy(lm_hbm, lm_v)
    pltpu.sync_copy(l0_hbm, l0_v)
    pltpu.sync_copy(l1_hbm, l1_v)

    lane = lax.iota(jnp.int32, _LANES)
    rows_per_ch = (v // 8) * 64

    def chunk_pair(ref, q):
        a = ref[pl.ds(q * _LANES, _LANES)]
        b = ref[pl.ds(min(q + _JH_CHUNKS, _N_CHUNKS - 1) * _LANES, _LANES)]
        return jnp.where(is_h1, b, a)

    for q in range(_JH_CHUNKS):
        lmj = chunk_pair(lm_v, q)
        r_idx = ch * rows_per_ch + (lmj >> 3) * 64 + nt * 8 + (lmj & 7)
        idx_v[pl.ds(q * _LANES, _LANES)] = r_idx

    pltpu.async_copy(m2_hbm.at[idx_v], vals_v, sem).wait()

    ones = jnp.ones((_LANES,), jnp.float32)
    zeros = jnp.zeros((_LANES,), jnp.float32)
    acc = zeros
    for g in range(_JH_CHUNKS):
        tvec = jnp.where(is_c1, chunk_pair(l1_v, g), chunk_pair(l0_v, g))
        w_a = jnp.where(g * _LANES + lane < _NUM_LM, 1.0, 0.0)
        w_b = jnp.where((g + _JH_CHUNKS) * _LANES + lane < _NUM_LM, 1.0, 0.0)
        wvec = jnp.where(is_h1, w_b, w_a)

        def row_step(r, carry):
            a_sq, a_cross = carry
            row = g * _LANES + r
            rsum = zeros
            rsq = zeros
            for k in range(_NB // _LANES):
                c = vals_v[row, pl.ds(k * _LANES, _LANES)]
                rsum = rsum + c
                rsq = rsq + c * c
            pos = jh * _ROWS + g * _LANES + r
            w = jnp.where(pos < _NUM_LM, ones, zeros)
            t = _splat(tvec, r)
            return a_sq + w * rsq, a_cross + (w * t) * rsum

        a_sq, a_cross = lax.fori_loop(0, _LANES, row_step, (zeros, zeros),
                                      unroll=False)
        acc = acc + a_sq - 2.0 * a_cross + wvec * (float(_NB) * tvec * tvec)

    acc_v[...] = acc
    pltpu.sync_copy(acc_v, out_hbm.at[wid])


def kernel(mapping, lm, landmarks):
    n, two, v = mapping.shape
    num_lm = lm.shape[0]

    m2 = (mapping
          .reshape(n // _NB, _NB, two, v // 8, 8)
          .transpose(2, 3, 0, 4, 1)
          .reshape(two * (v // 8) * (n // _NB) * 8, _NB))

    lm32 = lm.astype(jnp.int32)
    lm_pad = jnp.zeros((_LM_PAD,), jnp.int32).at[:num_lm].set(lm32)
    l0 = jnp.zeros((_LM_PAD,), jnp.float32).at[:num_lm].set(landmarks[:, 0])
    l1 = jnp.zeros((_LM_PAD,), jnp.float32).at[:num_lm].set(landmarks[:, 1])

    mesh = plsc.VectorSubcoreMesh(core_axis_name="c", subcore_axis_name="s",
                                  num_cores=_NC, num_subcores=_NS)
    partials = pl.kernel(
        functools.partial(_loss_kernel, n, v),
        out_type=jax.ShapeDtypeStruct((_NW, _LANES), jnp.float32),
        mesh=mesh,
        scratch_types=[
            pltpu.VMEM((_LM_PAD,), jnp.int32),
            pltpu.VMEM((_LM_PAD,), jnp.float32),
            pltpu.VMEM((_LM_PAD,), jnp.float32),
            pltpu.VMEM((_ROWS,), jnp.int32),
            pltpu.VMEM((_ROWS, _NB), jnp.float32),
            pltpu.VMEM((_LANES,), jnp.float32),
            pltpu.SemaphoreType.DMA,
        ],
    )(m2, lm_pad, l0, l1)
    return jnp.sum(partials) / n


# speedup vs baseline: 75.4955x; 1.0436x over previous
import functools

import jax
import jax.numpy as jnp
from jax import lax
from jax.experimental import pallas as pl
from jax.experimental.pallas import tpu as pltpu
from jax.experimental.pallas import tpu_sc as plsc

_NC = 2
_NS = 16
_NW = _NC * _NS

_LANES = 16
_NUM_LM = 68
_LM_PAD = 80
_N_CHUNKS = _LM_PAD // _LANES
_NB = 128
_JH_CHUNKS = 3
_PK = 2 * _LM_PAD

_GATHER_DNUMS = lax.GatherDimensionNumbers(
    offset_dims=(), collapsed_slice_dims=(0,), start_index_map=(0,))


def _splat(vec, idx_scalar):
    rr = jnp.full((_LANES, 1), idx_scalar, jnp.int32)
    return lax.gather(vec, rr, _GATHER_DNUMS, slice_sizes=(1,),
                      mode=lax.GatherScatterMode.PROMISE_IN_BOUNDS)


def _loss_kernel(n, v, m2_hbm, lm_hbm, pk_hbm, out_hbm,
                 lm_v, pk_v, idx0_v, idx1_v, idx2_v, v0_v, v1_v, v2_v,
                 acc_v, sem):
    idx_refs = (idx0_v, idx1_v, idx2_v)
    val_refs = (v0_v, v1_v, v2_v)
    wid = lax.axis_index("s") * _NC + lax.axis_index("c")
    nt = wid // 4
    ch = (wid // 2) % 2
    jh = wid % 2
    is_c1 = ch == 1
    is_h1 = jh == 1

    pltpu.sync_copy(lm_hbm, lm_v)
    pltpu.sync_copy(pk_hbm, pk_v)

    lane = lax.iota(jnp.int32, _LANES)
    rows_per_ch = (v // 8) * 64

    def chunk_pair(ref, base, q):
        a = ref[pl.ds(base + q * _LANES, _LANES)]
        b = ref[pl.ds(base + min(q + _JH_CHUNKS, _N_CHUNKS - 1) * _LANES,
                      _LANES)]
        return jnp.where(is_h1, b, a)

    for q in range(_JH_CHUNKS):
        lmj = chunk_pair(lm_v, 0, q)
        idx_refs[q][...] = (ch * rows_per_ch + (lmj >> 3) * 64
                            + nt * 8 + (lmj & 7))
        pltpu.async_copy(m2_hbm.at[idx_refs[q]], val_refs[q], sem)

    ones = jnp.ones((_LANES,), jnp.float32)
    zeros = jnp.zeros((_LANES,), jnp.float32)
    acc = zeros
    for g in range(_JH_CHUNKS):
        pltpu.make_async_copy(m2_hbm.at[idx_refs[g]], val_refs[g],
                              sem).wait()
        tvec = jnp.where(is_c1, chunk_pair(pk_v, _LM_PAD, g),
                         chunk_pair(pk_v, 0, g))
        vals = val_refs[g]

        def row_step(r, a):
            t = _splat(tvec, r)
            pos = jh * _JH_CHUNKS * _LANES + g * _LANES + r
            w = jnp.where(pos < _NUM_LM, ones, zeros)
            racc = zeros
            for k in range(_NB // _LANES):
                d = vals[r, pl.ds(k * _LANES, _LANES)] - t
                racc = racc + d * d
            return a + w * racc

        acc = lax.fori_loop(0, _LANES, row_step, acc, unroll=False)

    acc_v[...] = acc
    pltpu.sync_copy(acc_v, out_hbm.at[wid])


def kernel(mapping, lm, landmarks):
    n, two, v = mapping.shape
    num_lm = lm.shape[0]

    m2 = (mapping
          .reshape(n // _NB, _NB, two, v // 8, 8)
          .transpose(2, 3, 0, 4, 1)
          .reshape(two * (v // 8) * (n // _NB) * 8, _NB))

    lm_pad = jnp.zeros((_LM_PAD,), jnp.int32).at[:num_lm].set(
        lm.astype(jnp.int32))
    pk = jnp.zeros((2, _LM_PAD), jnp.float32)
    pk = pk.at[0, :num_lm].set(landmarks[:, 0])
    pk = pk.at[1, :num_lm].set(landmarks[:, 1])
    pk = pk.reshape(_PK)

    mesh = plsc.VectorSubcoreMesh(core_axis_name="c", subcore_axis_name="s",
                                  num_cores=_NC, num_subcores=_NS)
    partials = pl.kernel(
        functools.partial(_loss_kernel, n, v),
        out_type=jax.ShapeDtypeStruct((_NW, _LANES), jnp.float32),
        mesh=mesh,
        scratch_types=[
            pltpu.VMEM((_LM_PAD,), jnp.int32),
            pltpu.VMEM((_PK,), jnp.float32),
            pltpu.VMEM((_LANES,), jnp.int32),
            pltpu.VMEM((_LANES,), jnp.int32),
            pltpu.VMEM((_LANES,), jnp.int32),
            pltpu.VMEM((_LANES, _NB), jnp.float32),
            pltpu.VMEM((_LANES, _NB), jnp.float32),
            pltpu.VMEM((_LANES, _NB), jnp.float32),
            pltpu.VMEM((_LANES,), jnp.float32),
            pltpu.SemaphoreType.DMA,
        ],
    )(m2, lm_pad, pk)
    return jnp.sum(partials) / n


# speedup vs baseline: 75.5532x; 1.0008x over previous
import functools

import jax
import jax.numpy as jnp
from jax import lax
from jax.experimental import pallas as pl
from jax.experimental.pallas import tpu as pltpu
from jax.experimental.pallas import tpu_sc as plsc

_NC = 2
_NS = 16
_NW = _NC * _NS

_LANES = 16
_NUM_LM = 68
_LM_PAD = 80
_N_CHUNKS = _LM_PAD // _LANES
_NB = 128
_JH_CHUNKS = 3
_PK = 2 * _LM_PAD

_GATHER_DNUMS = lax.GatherDimensionNumbers(
    offset_dims=(), collapsed_slice_dims=(0,), start_index_map=(0,))


def _splat(vec, idx_scalar):
    rr = jnp.full((_LANES, 1), idx_scalar, jnp.int32)
    return lax.gather(vec, rr, _GATHER_DNUMS, slice_sizes=(1,),
                      mode=lax.GatherScatterMode.PROMISE_IN_BOUNDS)


def _loss_kernel(n, v, m2_hbm, lm_hbm, pk_hbm, out_hbm,
                 lm_v, pk_v, idx0_v, idx1_v, idx2_v, v0_v, v1_v, v2_v,
                 acc_v, sem):
    idx_refs = (idx0_v, idx1_v, idx2_v)
    val_refs = (v0_v, v1_v, v2_v)
    wid = lax.axis_index("s") * _NC + lax.axis_index("c")
    nt = wid // 4
    ch = (wid // 2) % 2
    jh = wid % 2
    is_c1 = ch == 1
    is_h1 = jh == 1

    pltpu.sync_copy(lm_hbm, lm_v)
    pltpu.sync_copy(pk_hbm, pk_v)

    lane = lax.iota(jnp.int32, _LANES)
    rows_per_ch = (v // 8) * 64

    def chunk_pair(ref, base, q):
        a = ref[pl.ds(base + q * _LANES, _LANES)]
        b = ref[pl.ds(base + min(q + _JH_CHUNKS, _N_CHUNKS - 1) * _LANES,
                      _LANES)]
        return jnp.where(is_h1, b, a)

    for q in range(_JH_CHUNKS):
        lmj = chunk_pair(lm_v, 0, q)
        idx_refs[q][...] = (ch * rows_per_ch + (lmj >> 3) * 64
                            + nt * 8 + (lmj & 7))
        pltpu.async_copy(m2_hbm.at[idx_refs[q]], val_refs[q], sem)

    ones = jnp.ones((_LANES,), jnp.float32)
    zeros = jnp.zeros((_LANES,), jnp.float32)
    acc = zeros
    for g in range(_JH_CHUNKS):
        pltpu.make_async_copy(m2_hbm.at[idx_refs[g]], val_refs[g],
                              sem).wait()
        tvec = jnp.where(is_c1, chunk_pair(pk_v, _LM_PAD, g),
                         chunk_pair(pk_v, 0, g))
        vals = val_refs[g]

        def row_step(r, a):
            t = _splat(tvec, r)
            pos = jh * _JH_CHUNKS * _LANES + g * _LANES + r
            w = jnp.where(pos < _NUM_LM, ones, zeros)
            def chunk_step(k, rc):
                d = vals[r, pl.ds(k * _LANES, _LANES)] - t
                return rc + d * d

            racc = lax.fori_loop(0, _NB // _LANES, chunk_step, zeros,
                                 unroll=False)
            return a + w * racc

        acc = lax.fori_loop(0, _LANES, row_step, acc, unroll=False)

    acc_v[...] = acc
    pltpu.sync_copy(acc_v, out_hbm.at[wid])


def kernel(mapping, lm, landmarks):
    n, two, v = mapping.shape
    num_lm = lm.shape[0]

    m2 = (mapping
          .reshape(n // _NB, _NB, two, v // 8, 8)
          .transpose(2, 3, 0, 4, 1)
          .reshape(two * (v // 8) * (n // _NB) * 8, _NB))

    lm_pad = jnp.zeros((_LM_PAD,), jnp.int32).at[:num_lm].set(
        lm.astype(jnp.int32))
    pk = jnp.zeros((2, _LM_PAD), jnp.float32)
    pk = pk.at[0, :num_lm].set(landmarks[:, 0])
    pk = pk.at[1, :num_lm].set(landmarks[:, 1])
    pk = pk.reshape(_PK)

    mesh = plsc.VectorSubcoreMesh(core_axis_name="c", subcore_axis_name="s",
                                  num_cores=_NC, num_subcores=_NS)
    partials = pl.kernel(
        functools.partial(_loss_kernel, n, v),
        out_type=jax.ShapeDtypeStruct((_NW, _LANES), jnp.float32),
        mesh=mesh,
        scratch_types=[
            pltpu.VMEM((_LM_PAD,), jnp.int32),
            pltpu.VMEM((_PK,), jnp.float32),
            pltpu.VMEM((_LANES,), jnp.int32),
            pltpu.VMEM((_LANES,), jnp.int32),
            pltpu.VMEM((_LANES,), jnp.int32),
            pltpu.VMEM((_LANES, _NB), jnp.float32),
            pltpu.VMEM((_LANES, _NB), jnp.float32),
            pltpu.VMEM((_LANES, _NB), jnp.float32),
            pltpu.VMEM((_LANES,), jnp.float32),
            pltpu.SemaphoreType.DMA,
        ],
    )(m2, lm_pad, pk)
    return jnp.sum(partials) / n
